# baseline (speedup 1.0000x reference)
import functools

import jax
import jax.numpy as jnp
from jax import lax
from jax.experimental import pallas as pl
from jax.experimental.pallas import tpu as pltpu
from jax.experimental.pallas import tpu_sc as plsc

N = 10000
E = 320000
D = 128
NEG_SLOPE = 0.2

NC = 2
NS = 16
NW = NC * NS
EPW = E // NW
CH = 80
NCHUNK = EPW // CH
NBUF = 3
PF = 2
RPS = 624
TAIL = N - NS * RPS


def _lin_kernel(x_ref, w_ref, ad_ref, as_ref, e1_ref, e2_ref,
                h_ref, sd_ref, ss_ref, pk1_ref, pk2_ref):
    h = jnp.dot(x_ref[...], w_ref[...], preferred_element_type=jnp.float32)
    h_ref[...] = h
    sd_ref[...] = jnp.sum(h * ad_ref[...], axis=1, keepdims=True)
    ss_ref[...] = jnp.sum(h * as_ref[...], axis=1, keepdims=True)
    e1 = e1_ref[...]
    e2 = e2_ref[...]
    pk1_ref[...] = (e1[1:2] << 16) | e1[0:1]
    pk2_ref[...] = (e2[1:2] << 16) | e2[0:1]


def _lin(x, w, a_dst, a_src, e1, e2):
    return pl.pallas_call(
        _lin_kernel,
        out_shape=[
            jax.ShapeDtypeStruct((N, D), jnp.float32),
            jax.ShapeDtypeStruct((N, 1), jnp.float32),
            jax.ShapeDtypeStruct((N, 1), jnp.float32),
            jax.ShapeDtypeStruct((1, E), jnp.int32),
            jax.ShapeDtypeStruct((1, E), jnp.int32),
        ],
    )(x, w, a_dst, a_src, e1, e2)


def _combine_lin_kernel(up_ref, sp_ref, h_ref, sd_ref, ss_ref, w_ref,
                        ad_ref, as_ref, h2_ref, sd2_ref, ss2_ref):
    wl = jnp.exp(jax.nn.leaky_relu(sd_ref[...] + ss_ref[...], NEG_SLOPE))
    u = up_ref[0] + up_ref[1] + wl * h_ref[...]
    s = sp_ref[0] + sp_ref[1] + wl + 1e-16
    g = jnp.maximum(u / s, 0.0)
    h2 = jnp.dot(g, w_ref[...], preferred_element_type=jnp.float32)
    h2_ref[...] = h2
    sd2_ref[...] = jnp.sum(h2 * ad_ref[...], axis=1, keepdims=True)
    ss2_ref[...] = jnp.sum(h2 * as_ref[...], axis=1, keepdims=True)


def _combine_lin(u_parts, s_parts, h, sd, ss, w, a_dst, a_src):
    return pl.pallas_call(
        _combine_lin_kernel,
        out_shape=[
            jax.ShapeDtypeStruct((N, D), jnp.float32),
            jax.ShapeDtypeStruct((N, 1), jnp.float32),
            jax.ShapeDtypeStruct((N, 1), jnp.float32),
        ],
    )(u_parts, s_parts.reshape(NC, N, 1), h, sd, ss, w, a_dst, a_src)


def _final_kernel(up_ref, sp_ref, h_ref, sd_ref, ss_ref, o_ref):
    wl = jnp.exp(jax.nn.leaky_relu(sd_ref[...] + ss_ref[...], NEG_SLOPE))
    u = up_ref[0] + up_ref[1] + wl * h_ref[...]
    s = sp_ref[0] + sp_ref[1] + wl + 1e-16
    o = u / s
    m = jnp.max(o, axis=1, keepdims=True)
    z = jnp.log(jnp.sum(jnp.exp(o - m), axis=1, keepdims=True))
    o_ref[...] = o - m - z


def _final(u_parts, s_parts, h, sd, ss):
    return pl.pallas_call(
        _final_kernel,
        out_shape=jax.ShapeDtypeStruct((N, D), jnp.float32),
    )(u_parts, s_parts.reshape(NC, N, 1), h, sd, ss)


_MESH = plsc.VectorSubcoreMesh(
    core_axis_name="c", subcore_axis_name="s", num_cores=NC, num_subcores=NS
)


@functools.partial(
    pl.kernel,
    out_type=[
        jax.ShapeDtypeStruct((NC, N, D), jnp.float32),
        jax.ShapeDtypeStruct((NC, N), jnp.float32),
    ],
    mesh=_MESH,
    compiler_params=pltpu.CompilerParams(
        needs_layout_passes=False, use_tc_tiling_on_sc=False
    ),
    scratch_types=[
        pltpu.VMEM((NCHUNK, CH), jnp.int32),
        pltpu.VMEM((NBUF, CH), jnp.int32),
        pltpu.VMEM((NBUF, CH), jnp.int32),
        pltpu.VMEM((NBUF, CH), jnp.float32),
        pltpu.VMEM((NBUF, CH), jnp.float32),
        pltpu.VMEM((NBUF, CH), jnp.float32),
        pltpu.VMEM((NBUF, CH, D), jnp.float32),
        pltpu.VMEM_SHARED((N, D), jnp.float32),
        pltpu.VMEM_SHARED((N,), jnp.float32),
        pltpu.SemaphoreType.DMA((NBUF,)),
        pltpu.SemaphoreType.DMA((NBUF,)),
        pltpu.SemaphoreType.DMA((NBUF,)),
        pltpu.SemaphoreType.DMA((NBUF,)),
    ],
)
def _edge_kernel(pk_hbm, h_hbm, ssrc_hbm, sdst_hbm, zu_hbm, zs_hbm,
                 u_out, s_out, pk_v, idxs, idxd, scs2, scd2, w2, rows2,
                 u_sh, s_sh, gsem, sgsem, usem, ssem):
    cid = lax.axis_index("c")
    sid = lax.axis_index("s")
    wid = sid * NC + cid

    pltpu.sync_copy(zu_hbm.at[pl.ds(sid * RPS, RPS)],
                    u_sh.at[pl.ds(sid * RPS, RPS)])

    @pl.when(sid == 0)
    def _():
        pltpu.sync_copy(zu_hbm.at[pl.ds(NS * RPS, TAIL)],
                        u_sh.at[pl.ds(NS * RPS, TAIL)])
        pltpu.sync_copy(zs_hbm, s_sh)

    pltpu.sync_copy(pk_hbm.at[wid], pk_v)
    plsc.subcore_barrier()

    def unpack(kk, bslot):
        for j in range(CH // 16):
            sl = pl.ds(j * 16, 16)
            p16 = pk_v[kk, sl]
            idxs[bslot, sl] = lax.bitwise_and(p16, 0xFFFF)
            idxd[bslot, sl] = lax.shift_right_logical(p16, 16)

    def start_gathers(bslot):
        pltpu.make_async_copy(
            h_hbm.at[idxs.at[bslot]], rows2.at[bslot], gsem.at[bslot]).start()
        pltpu.make_async_copy(
            ssrc_hbm.at[idxs.at[bslot]], scs2.at[bslot],
            sgsem.at[bslot]).start()
        pltpu.make_async_copy(
            sdst_hbm.at[idxd.at[bslot]], scd2.at[bslot],
            sgsem.at[bslot]).start()

    for b in range(PF):
        unpack(b, b)
        start_gathers(b)

    def chunk_body(k, carry):
        bb = lax.rem(k, NBUF)
        pltpu.make_async_copy(
            h_hbm.at[idxs.at[bb]], rows2.at[bb], gsem.at[bb]).wait()
        pltpu.make_async_copy(
            ssrc_hbm.at[idxs.at[bb]], scs2.at[bb], sgsem.at[bb]).wait()
        pltpu.make_async_copy(
            sdst_hbm.at[idxd.at[bb]], scd2.at[bb], sgsem.at[bb]).wait()

        for j in range(CH // 16):
            sl = pl.ds(j * 16, 16)
            aa = scs2[bb, sl] + scd2[bb, sl]
            aa = jnp.where(aa > 0, aa, NEG_SLOPE * aa)
            w2[bb, sl] = jnp.exp(aa)
        pltpu.make_async_copy(
            w2.at[bb], s_sh.at[idxd.at[bb]], ssem.at[bb]).start(add=True)

        for g in range(CH // 16):
            w16 = w2[bb, pl.ds(g * 16, 16)]
            for l in range(16):
                wv = jnp.full((16,), w16[l], jnp.float32)
                e = g * 16 + l
                for c in range(D // 16):
                    cs = pl.ds(c * 16, 16)
                    rows2[bb, e, cs] = rows2[bb, e, cs] * wv
        pltpu.make_async_copy(
            rows2.at[bb], u_sh.at[idxd.at[bb]], usem.at[bb]).start(add=True)

        @pl.when(k + PF < NCHUNK)
        def _():
            bp = lax.rem(k + PF, NBUF)

            @pl.when(k >= NBUF - PF)
            def _():
                pltpu.make_async_copy(
                    rows2.at[bp], u_sh.at[idxd.at[bp]], usem.at[bp]).wait()
                pltpu.make_async_copy(
                    w2.at[bp], s_sh.at[idxd.at[bp]], ssem.at[bp]).wait()
            unpack(k + PF, bp)
            start_gathers(bp)
        return carry

    lax.fori_loop(0, NCHUNK, chunk_body, 0)
    for b in range(NBUF):
        pltpu.make_async_copy(
            rows2.at[b], u_sh.at[idxd.at[b]], usem.at[b]).wait()
        pltpu.make_async_copy(
            w2.at[b], s_sh.at[idxd.at[b]], ssem.at[b]).wait()
    plsc.subcore_barrier()

    pltpu.sync_copy(u_sh.at[pl.ds(sid * RPS, RPS)],
                    u_out.at[cid, pl.ds(sid * RPS, RPS)])

    @pl.when(sid == 0)
    def _():
        pltpu.sync_copy(u_sh.at[pl.ds(NS * RPS, TAIL)],
                        u_out.at[cid, pl.ds(NS * RPS, TAIL)])
        pltpu.sync_copy(s_sh, s_out.at[cid])


def kernel(x, edge_index1, edge_index2, W1, a1, W2, a2):
    zu = jnp.zeros((N, D), jnp.float32)
    zs = jnp.zeros((N,), jnp.float32)

    a1d = a1[:D].reshape(1, D)
    a1s = a1[D:].reshape(1, D)
    a2d = a2[:D].reshape(1, D)
    a2s = a2[D:].reshape(1, D)

    h1, s1d, s1s, pk1, pk2 = _lin(x, W1, a1d, a1s, edge_index1, edge_index2)
    pk1 = pk1.reshape(NW, NCHUNK, CH)
    pk2 = pk2.reshape(NW, NCHUNK, CH)
    u1, s1 = _edge_kernel(pk1, h1, s1s.reshape(N), s1d.reshape(N), zu, zs)
    h2, s2d, s2s = _combine_lin(u1, s1, h1, s1d, s1s, W2, a2d, a2s)
    u2, s2 = _edge_kernel(pk2, h2, s2s.reshape(N), s2d.reshape(N), zu, zs)
    return _final(u2, s2, h2, s2d, s2s)


# speedup vs baseline: 1.0068x; 1.0068x over previous
import functools

import jax
import jax.numpy as jnp
from jax import lax
from jax.experimental import pallas as pl
from jax.experimental.pallas import tpu as pltpu
from jax.experimental.pallas import tpu_sc as plsc

N = 10000
E = 320000
D = 128
NEG_SLOPE = 0.2

NC = 2
NS = 16
NW = NC * NS
EPW = E // NW
CH = 80
NCHUNK = EPW // CH
NBUF = 3
PF = 2
RPS = 624
TAIL = N - NS * RPS


def _lin_kernel(x_ref, w_ref, ad_ref, as_ref, e1_ref, e2_ref,
                h_ref, sd_ref, ss_ref, pk1_ref, pk2_ref):
    h = jnp.dot(x_ref[...], w_ref[...], preferred_element_type=jnp.float32)
    h_ref[...] = h
    sd_ref[...] = jnp.sum(h * ad_ref[...], axis=1, keepdims=True)
    ss_ref[...] = jnp.sum(h * as_ref[...], axis=1, keepdims=True)
    e1 = e1_ref[...]
    e2 = e2_ref[...]
    pk1_ref[...] = (e1[1:2] << 16) | e1[0:1]
    pk2_ref[...] = (e2[1:2] << 16) | e2[0:1]


def _lin(x, w, a_dst, a_src, e1, e2):
    return pl.pallas_call(
        _lin_kernel,
        out_shape=[
            jax.ShapeDtypeStruct((N, D), jnp.float32),
            jax.ShapeDtypeStruct((N, 1), jnp.float32),
            jax.ShapeDtypeStruct((N, 1), jnp.float32),
            jax.ShapeDtypeStruct((1, E), jnp.int32),
            jax.ShapeDtypeStruct((1, E), jnp.int32),
        ],
    )(x, w, a_dst, a_src, e1, e2)


def _combine_lin_kernel(up_ref, sp_ref, h_ref, sd_ref, ss_ref, w_ref,
                        ad_ref, as_ref, h2_ref, sd2_ref, ss2_ref):
    wl = jnp.exp(jax.nn.leaky_relu(sd_ref[...] + ss_ref[...], NEG_SLOPE))
    u = up_ref[0] + up_ref[1] + wl * h_ref[...]
    s = sp_ref[0] + sp_ref[1] + wl + 1e-16
    g = jnp.maximum(u / s, 0.0)
    h2 = jnp.dot(g, w_ref[...], preferred_element_type=jnp.float32)
    h2_ref[...] = h2
    sd2_ref[...] = jnp.sum(h2 * ad_ref[...], axis=1, keepdims=True)
    ss2_ref[...] = jnp.sum(h2 * as_ref[...], axis=1, keepdims=True)


def _combine_lin(u_parts, s_parts, h, sd, ss, w, a_dst, a_src):
    return pl.pallas_call(
        _combine_lin_kernel,
        out_shape=[
            jax.ShapeDtypeStruct((N, D), jnp.float32),
            jax.ShapeDtypeStruct((N, 1), jnp.float32),
            jax.ShapeDtypeStruct((N, 1), jnp.float32),
        ],
    )(u_parts, s_parts.reshape(NC, N, 1), h, sd, ss, w, a_dst, a_src)


def _final_kernel(up_ref, sp_ref, h_ref, sd_ref, ss_ref, o_ref):
    wl = jnp.exp(jax.nn.leaky_relu(sd_ref[...] + ss_ref[...], NEG_SLOPE))
    u = up_ref[0] + up_ref[1] + wl * h_ref[...]
    s = sp_ref[0] + sp_ref[1] + wl + 1e-16
    o = u / s
    m = jnp.max(o, axis=1, keepdims=True)
    z = jnp.log(jnp.sum(jnp.exp(o - m), axis=1, keepdims=True))
    o_ref[...] = o - m - z


def _final(u_parts, s_parts, h, sd, ss):
    return pl.pallas_call(
        _final_kernel,
        out_shape=jax.ShapeDtypeStruct((N, D), jnp.float32),
    )(u_parts, s_parts.reshape(NC, N, 1), h, sd, ss)


_MESH = plsc.VectorSubcoreMesh(
    core_axis_name="c", subcore_axis_name="s", num_cores=NC, num_subcores=NS
)


@functools.partial(
    pl.kernel,
    out_type=[
        jax.ShapeDtypeStruct((NC, N, D), jnp.float32),
        jax.ShapeDtypeStruct((NC, N), jnp.float32),
    ],
    mesh=_MESH,
    compiler_params=pltpu.CompilerParams(
        needs_layout_passes=False, use_tc_tiling_on_sc=False
    ),
    scratch_types=[
        pltpu.VMEM((NCHUNK, CH), jnp.int32),
        pltpu.VMEM((NBUF, CH), jnp.int32),
        pltpu.VMEM((NBUF, CH), jnp.int32),
        pltpu.VMEM((NBUF, CH), jnp.int32),
        pltpu.VMEM((NBUF, CH), jnp.float32),
        pltpu.VMEM((NBUF, CH), jnp.float32),
        pltpu.VMEM((NBUF, CH), jnp.float32),
        pltpu.VMEM((NBUF, CH, D), jnp.float32),
        pltpu.VMEM_SHARED((N, D), jnp.float32),
        pltpu.VMEM_SHARED((N,), jnp.float32),
        pltpu.SemaphoreType.DMA((NBUF,)),
        pltpu.SemaphoreType.DMA((NBUF,)),
        pltpu.SemaphoreType.DMA((NBUF,)),
        pltpu.SemaphoreType.DMA((NBUF,)),
    ],
)
def _edge_kernel(pk_hbm, h_hbm, ssrc_hbm, sdst_hbm, zu_hbm, zs_hbm,
                 u_out, s_out, pk_v, idxs, idxd, sidx, scs2, scd2, w2, rows2,
                 u_sh, s_sh, gsem, sgsem, usem, ssem):
    cid = lax.axis_index("c")
    sid = lax.axis_index("s")
    wid = sid * NC + cid

    pltpu.sync_copy(zu_hbm.at[pl.ds(sid * RPS, RPS)],
                    u_sh.at[pl.ds(sid * RPS, RPS)])

    @pl.when(sid == 0)
    def _():
        pltpu.sync_copy(zu_hbm.at[pl.ds(NS * RPS, TAIL)],
                        u_sh.at[pl.ds(NS * RPS, TAIL)])
        pltpu.sync_copy(zs_hbm, s_sh)

    pltpu.sync_copy(pk_hbm.at[wid], pk_v)
    plsc.subcore_barrier()

    def unpack(kk, bslot):
        for j in range(CH // 16):
            sl = pl.ds(j * 16, 16)
            p16 = pk_v[kk, sl]
            idxs[bslot, sl] = lax.bitwise_and(p16, 0xFFFF)
            idxd[bslot, sl] = lax.shift_right_logical(p16, 16)

    def start_gathers(bslot):
        pltpu.make_async_copy(
            h_hbm.at[idxs.at[bslot]], rows2.at[bslot], gsem.at[bslot]).start()
        pltpu.make_async_copy(
            ssrc_hbm.at[idxs.at[bslot]], scs2.at[bslot],
            sgsem.at[bslot]).start()
        pltpu.make_async_copy(
            sdst_hbm.at[idxd.at[bslot]], scd2.at[bslot],
            sgsem.at[bslot]).start()

    for b in range(PF):
        unpack(b, b)
        start_gathers(b)

    def chunk_body(k, carry):
        bb = lax.rem(k, NBUF)
        pltpu.make_async_copy(
            h_hbm.at[idxs.at[bb]], rows2.at[bb], gsem.at[bb]).wait()
        pltpu.make_async_copy(
            ssrc_hbm.at[idxs.at[bb]], scs2.at[bb], sgsem.at[bb]).wait()
        pltpu.make_async_copy(
            sdst_hbm.at[idxd.at[bb]], scd2.at[bb], sgsem.at[bb]).wait()

        @pl.when(k >= NBUF)
        def _():
            pltpu.make_async_copy(
                w2.at[bb], s_sh.at[sidx.at[bb]], ssem.at[bb]).wait()

        for j in range(CH // 16):
            sl = pl.ds(j * 16, 16)
            aa = scs2[bb, sl] + scd2[bb, sl]
            aa = jnp.where(aa > 0, aa, NEG_SLOPE * aa)
            w2[bb, sl] = jnp.exp(aa)
            sidx[bb, sl] = idxd[bb, sl]
        pltpu.make_async_copy(
            w2.at[bb], s_sh.at[sidx.at[bb]], ssem.at[bb]).start(add=True)

        for g in range(CH // 16):
            w16 = w2[bb, pl.ds(g * 16, 16)]
            for l in range(16):
                wv = jnp.full((16,), w16[l], jnp.float32)
                e = g * 16 + l
                for c in range(D // 16):
                    cs = pl.ds(c * 16, 16)
                    rows2[bb, e, cs] = rows2[bb, e, cs] * wv
        pltpu.make_async_copy(
            rows2.at[bb], u_sh.at[idxd.at[bb]], usem.at[bb]).start(add=True)

        @pl.when(k + PF < NCHUNK)
        def _():
            bp = lax.rem(k + PF, NBUF)

            @pl.when(k >= NBUF - PF)
            def _():
                pltpu.make_async_copy(
                    rows2.at[bp], u_sh.at[idxd.at[bp]], usem.at[bp]).wait()
            unpack(k + PF, bp)
            start_gathers(bp)
        return carry

    lax.fori_loop(0, NCHUNK, chunk_body, 0)
    for b in range(NBUF):
        pltpu.make_async_copy(
            rows2.at[b], u_sh.at[idxd.at[b]], usem.at[b]).wait()
        pltpu.make_async_copy(
            w2.at[b], s_sh.at[sidx.at[b]], ssem.at[b]).wait()
    plsc.subcore_barrier()

    pltpu.sync_copy(u_sh.at[pl.ds(sid * RPS, RPS)],
                    u_out.at[cid, pl.ds(sid * RPS, RPS)])

    @pl.when(sid == 0)
    def _():
        pltpu.sync_copy(u_sh.at[pl.ds(NS * RPS, TAIL)],
                        u_out.at[cid, pl.ds(NS * RPS, TAIL)])
        pltpu.sync_copy(s_sh, s_out.at[cid])


def kernel(x, edge_index1, edge_index2, W1, a1, W2, a2):
    zu = jnp.zeros((N, D), jnp.float32)
    zs = jnp.zeros((N,), jnp.float32)

    a1d = a1[:D].reshape(1, D)
    a1s = a1[D:].reshape(1, D)
    a2d = a2[:D].reshape(1, D)
    a2s = a2[D:].reshape(1, D)

    h1, s1d, s1s, pk1, pk2 = _lin(x, W1, a1d, a1s, edge_index1, edge_index2)
    pk1 = pk1.reshape(NW, NCHUNK, CH)
    pk2 = pk2.reshape(NW, NCHUNK, CH)
    u1, s1 = _edge_kernel(pk1, h1, s1s.reshape(N), s1d.reshape(N), zu, zs)
    h2, s2d, s2s = _combine_lin(u1, s1, h1, s1d, s1s, W2, a2d, a2s)
    u2, s2 = _edge_kernel(pk2, h2, s2s.reshape(N), s2d.reshape(N), zu, zs)
    return _final(u2, s2, h2, s2d, s2s)


# speedup vs baseline: 1.0163x; 1.0095x over previous
import functools

import jax
import jax.numpy as jnp
from jax import lax
from jax.experimental import pallas as pl
from jax.experimental.pallas import tpu as pltpu
from jax.experimental.pallas import tpu_sc as plsc

N = 10000
E = 320000
D = 128
NEG_SLOPE = 0.2

NC = 2
NS = 16
NW = NC * NS
EPW = E // NW
CH = 80
NCHUNK = EPW // CH
NBUF = 3
PF = 2
RPS = 624
TAIL = N - NS * RPS


def _lin_kernel(x_ref, w_ref, ad_ref, as_ref, h_ref, sd_ref, ss_ref):
    h = jnp.dot(x_ref[...], w_ref[...], preferred_element_type=jnp.float32)
    h_ref[...] = h
    sd_ref[...] = jnp.sum(h * ad_ref[...], axis=1, keepdims=True)
    ss_ref[...] = jnp.sum(h * as_ref[...], axis=1, keepdims=True)


def _lin(x, w, a_dst, a_src):
    return pl.pallas_call(
        _lin_kernel,
        out_shape=[
            jax.ShapeDtypeStruct((N, D), jnp.float32),
            jax.ShapeDtypeStruct((N, 1), jnp.float32),
            jax.ShapeDtypeStruct((N, 1), jnp.float32),
        ],
    )(x, w, a_dst, a_src)


def _combine_lin_kernel(up_ref, sp_ref, h_ref, sd_ref, ss_ref, w_ref,
                        ad_ref, as_ref, h2_ref, sd2_ref, ss2_ref):
    wl = jnp.exp(jax.nn.leaky_relu(sd_ref[...] + ss_ref[...], NEG_SLOPE))
    u = up_ref[0] + up_ref[1] + wl * h_ref[...]
    s = sp_ref[0] + sp_ref[1] + wl + 1e-16
    g = jnp.maximum(u / s, 0.0)
    h2 = jnp.dot(g, w_ref[...], preferred_element_type=jnp.float32)
    h2_ref[...] = h2
    sd2_ref[...] = jnp.sum(h2 * ad_ref[...], axis=1, keepdims=True)
    ss2_ref[...] = jnp.sum(h2 * as_ref[...], axis=1, keepdims=True)


def _combine_lin(u_parts, s_parts, h, sd, ss, w, a_dst, a_src):
    return pl.pallas_call(
        _combine_lin_kernel,
        out_shape=[
            jax.ShapeDtypeStruct((N, D), jnp.float32),
            jax.ShapeDtypeStruct((N, 1), jnp.float32),
            jax.ShapeDtypeStruct((N, 1), jnp.float32),
        ],
    )(u_parts, s_parts.reshape(NC, N, 1), h, sd, ss, w, a_dst, a_src)


def _final_kernel(up_ref, sp_ref, h_ref, sd_ref, ss_ref, o_ref):
    wl = jnp.exp(jax.nn.leaky_relu(sd_ref[...] + ss_ref[...], NEG_SLOPE))
    u = up_ref[0] + up_ref[1] + wl * h_ref[...]
    s = sp_ref[0] + sp_ref[1] + wl + 1e-16
    o = u / s
    m = jnp.max(o, axis=1, keepdims=True)
    z = jnp.log(jnp.sum(jnp.exp(o - m), axis=1, keepdims=True))
    o_ref[...] = o - m - z


def _final(u_parts, s_parts, h, sd, ss):
    return pl.pallas_call(
        _final_kernel,
        out_shape=jax.ShapeDtypeStruct((N, D), jnp.float32),
    )(u_parts, s_parts.reshape(NC, N, 1), h, sd, ss)


_MESH = plsc.VectorSubcoreMesh(
    core_axis_name="c", subcore_axis_name="s", num_cores=NC, num_subcores=NS
)


@functools.partial(
    pl.kernel,
    out_type=[
        jax.ShapeDtypeStruct((NC, N, D), jnp.float32),
        jax.ShapeDtypeStruct((NC, N), jnp.float32),
    ],
    mesh=_MESH,
    compiler_params=pltpu.CompilerParams(
        needs_layout_passes=False, use_tc_tiling_on_sc=False
    ),
    scratch_types=[
        pltpu.VMEM((NCHUNK, CH), jnp.int32),
        pltpu.VMEM((NBUF, CH), jnp.int32),
        pltpu.VMEM((NBUF, CH), jnp.int32),
        pltpu.VMEM((NBUF, CH), jnp.int32),
        pltpu.VMEM((NBUF, CH), jnp.float32),
        pltpu.VMEM((NBUF, CH), jnp.float32),
        pltpu.VMEM((NBUF, CH), jnp.float32),
        pltpu.VMEM((NBUF, CH, D), jnp.float32),
        pltpu.VMEM_SHARED((N, D), jnp.float32),
        pltpu.VMEM_SHARED((N,), jnp.float32),
        pltpu.SemaphoreType.DMA((NBUF,)),
        pltpu.SemaphoreType.DMA((NBUF,)),
        pltpu.SemaphoreType.DMA((NBUF,)),
        pltpu.SemaphoreType.DMA((NBUF,)),
    ],
)
def _edge_kernel(pk_hbm, h_hbm, ssrc_hbm, sdst_hbm, zu_hbm, zs_hbm,
                 u_out, s_out, pk_v, idxs, idxd, sidx, scs2, scd2, w2, rows2,
                 u_sh, s_sh, gsem, sgsem, usem, ssem):
    cid = lax.axis_index("c")
    sid = lax.axis_index("s")
    wid = sid * NC + cid

    pltpu.sync_copy(zu_hbm.at[pl.ds(sid * RPS, RPS)],
                    u_sh.at[pl.ds(sid * RPS, RPS)])

    @pl.when(sid == 0)
    def _():
        pltpu.sync_copy(zu_hbm.at[pl.ds(NS * RPS, TAIL)],
                        u_sh.at[pl.ds(NS * RPS, TAIL)])
        pltpu.sync_copy(zs_hbm, s_sh)

    pltpu.sync_copy(pk_hbm.at[wid], pk_v)
    plsc.subcore_barrier()

    def unpack(kk, bslot):
        for j in range(CH // 16):
            sl = pl.ds(j * 16, 16)
            p16 = pk_v[kk, sl]
            idxs[bslot, sl] = lax.bitwise_and(p16, 0xFFFF)
            idxd[bslot, sl] = lax.shift_right_logical(p16, 16)

    def start_gathers(bslot):
        pltpu.make_async_copy(
            h_hbm.at[idxs.at[bslot]], rows2.at[bslot], gsem.at[bslot]).start()
        pltpu.make_async_copy(
            ssrc_hbm.at[idxs.at[bslot]], scs2.at[bslot],
            sgsem.at[bslot]).start()
        pltpu.make_async_copy(
            sdst_hbm.at[idxd.at[bslot]], scd2.at[bslot],
            sgsem.at[bslot]).start()

    for b in range(PF):
        unpack(b, b)
        start_gathers(b)

    def chunk_body(k, carry):
        bb = lax.rem(k, NBUF)
        pltpu.make_async_copy(
            h_hbm.at[idxs.at[bb]], rows2.at[bb], gsem.at[bb]).wait()
        pltpu.make_async_copy(
            ssrc_hbm.at[idxs.at[bb]], scs2.at[bb], sgsem.at[bb]).wait()
        pltpu.make_async_copy(
            sdst_hbm.at[idxd.at[bb]], scd2.at[bb], sgsem.at[bb]).wait()

        @pl.when(k >= NBUF)
        def _():
            pltpu.make_async_copy(
                w2.at[bb], s_sh.at[sidx.at[bb]], ssem.at[bb]).wait()

        for j in range(CH // 16):
            sl = pl.ds(j * 16, 16)
            aa = scs2[bb, sl] + scd2[bb, sl]
            aa = jnp.where(aa > 0, aa, NEG_SLOPE * aa)
            w2[bb, sl] = jnp.exp(aa)
            sidx[bb, sl] = idxd[bb, sl]
        pltpu.make_async_copy(
            w2.at[bb], s_sh.at[sidx.at[bb]], ssem.at[bb]).start(add=True)

        for g in range(CH // 16):
            w16 = w2[bb, pl.ds(g * 16, 16)]
            for l in range(16):
                wv = jnp.full((16,), w16[l], jnp.float32)
                e = g * 16 + l
                for c in range(D // 16):
                    cs = pl.ds(c * 16, 16)
                    rows2[bb, e, cs] = rows2[bb, e, cs] * wv
        pltpu.make_async_copy(
            rows2.at[bb], u_sh.at[idxd.at[bb]], usem.at[bb]).start(add=True)

        @pl.when(k + PF < NCHUNK)
        def _():
            bp = lax.rem(k + PF, NBUF)

            @pl.when(k >= NBUF - PF)
            def _():
                pltpu.make_async_copy(
                    rows2.at[bp], u_sh.at[idxd.at[bp]], usem.at[bp]).wait()
            unpack(k + PF, bp)
            start_gathers(bp)
        return carry

    lax.fori_loop(0, NCHUNK, chunk_body, 0)
    for b in range(NBUF):
        pltpu.make_async_copy(
            rows2.at[b], u_sh.at[idxd.at[b]], usem.at[b]).wait()
        pltpu.make_async_copy(
            w2.at[b], s_sh.at[sidx.at[b]], ssem.at[b]).wait()
    plsc.subcore_barrier()

    pltpu.sync_copy(u_sh.at[pl.ds(sid * RPS, RPS)],
                    u_out.at[cid, pl.ds(sid * RPS, RPS)])

    @pl.when(sid == 0)
    def _():
        pltpu.sync_copy(u_sh.at[pl.ds(NS * RPS, TAIL)],
                        u_out.at[cid, pl.ds(NS * RPS, TAIL)])
        pltpu.sync_copy(s_sh, s_out.at[cid])


def kernel(x, edge_index1, edge_index2, W1, a1, W2, a2):
    zu = jnp.zeros((N, D), jnp.float32)
    zs = jnp.zeros((N,), jnp.float32)

    a1d = a1[:D].reshape(1, D)
    a1s = a1[D:].reshape(1, D)
    a2d = a2[:D].reshape(1, D)
    a2s = a2[D:].reshape(1, D)

    pk1 = ((edge_index1[1] << 16) | edge_index1[0]).reshape(NW, NCHUNK, CH)
    pk2 = ((edge_index2[1] << 16) | edge_index2[0]).reshape(NW, NCHUNK, CH)

    h1, s1d, s1s = _lin(x, W1, a1d, a1s)
    u1, s1 = _edge_kernel(pk1, h1, s1s.reshape(N), s1d.reshape(N), zu, zs)
    h2, s2d, s2s = _combine_lin(u1, s1, h1, s1d, s1s, W2, a2d, a2s)
    u2, s2 = _edge_kernel(pk2, h2, s2s.reshape(N), s2d.reshape(N), zu, zs)
    return _final(u2, s2, h2, s2d, s2s)
